# baseline (speedup 1.0000x reference)
import functools

import jax
import jax.numpy as jnp
from jax import lax
from jax.experimental import pallas as pl
from jax.experimental.pallas import tpu as pltpu
from jax.experimental.pallas import tpu_sc as plsc

_B = 16
_T = 256
_ND = 4
_NC = 6
_BINS = 256
_SA = 8
_D = 1024
_NTOK = _B * _T

_NSLICE = 4
_SLICE = _NTOK // _NSLICE

_SC_CORES = 2
_SC_SUBCORES = 16
_NW = _SC_CORES * _SC_SUBCORES
_CH = 8
_RPC = _CH * _ND


def _make_sc_gather_sum(ntok):
    tpw = ntok // _NW
    nchunk = tpw // _CH
    mesh = plsc.VectorSubcoreMesh(core_axis_name="c", subcore_axis_name="s")

    @functools.partial(
        pl.kernel,
        mesh=mesh,
        out_type=jax.ShapeDtypeStruct((ntok, _D), jnp.float32),
        scratch_types=[
            pltpu.VMEM((tpw * _ND,), jnp.int32),
            pltpu.VMEM((_RPC, _D), jnp.float32),
            pltpu.VMEM((_RPC, _D), jnp.float32),
            pltpu.VMEM((_CH, _D), jnp.float32),
            pltpu.VMEM((_CH, _D), jnp.float32),
            pltpu.SemaphoreType.DMA,
            pltpu.SemaphoreType.DMA,
            pltpu.SemaphoreType.DMA,
            pltpu.SemaphoreType.DMA,
        ],
    )
    def gather_sum(table_hbm, idx_hbm, out_hbm, idx_v, buf_a, buf_b,
                   acc_a, acc_b, sem_a, sem_b, sem_oa, sem_ob):
        wid = lax.axis_index("s") * _SC_CORES + lax.axis_index("c")
        tok0 = wid * tpw
        pltpu.sync_copy(idx_hbm.at[pl.ds(tok0 * _ND, tpw * _ND)], idx_v)

        def compute(buf, acc):
            @plsc.parallel_loop(0, _CH * 16, 1, unroll=4)
            def cbody(i):
                t = i >> 4
                dd = i & 15
                for u in range(4):
                    sl = pl.ds(dd * 64 + u * 16, 16)
                    acc[t, sl] = ((buf[4 * t + 0, sl] + buf[4 * t + 1, sl])
                                  + (buf[4 * t + 2, sl] + buf[4 * t + 3, sl]))

        def wait_gather(buf, sem):
            pltpu.make_async_copy(
                table_hbm.at[idx_v.at[pl.ds(0, _RPC)]], buf, sem).wait()

        def wait_out(acc, sem):
            pltpu.make_async_copy(
                acc, out_hbm.at[pl.ds(tok0, _CH)], sem).wait()

        pltpu.async_copy(table_hbm.at[idx_v.at[pl.ds(0, _RPC)]], buf_a, sem_a)

        def pbody(p, carry):
            c0 = 2 * p
            pltpu.async_copy(
                table_hbm.at[idx_v.at[pl.ds((c0 + 1) * _RPC, _RPC)]],
                buf_b, sem_b)
            wait_gather(buf_a, sem_a)

            @pl.when(p > 0)
            def _():
                wait_out(acc_a, sem_oa)
            compute(buf_a, acc_a)
            pltpu.async_copy(
                acc_a, out_hbm.at[pl.ds(tok0 + c0 * _CH, _CH)], sem_oa)

            @pl.when(p + 1 < nchunk // 2)
            def _():
                pltpu.async_copy(
                    table_hbm.at[idx_v.at[pl.ds((c0 + 2) * _RPC, _RPC)]],
                    buf_a, sem_a)
            wait_gather(buf_b, sem_b)

            @pl.when(p > 0)
            def _():
                wait_out(acc_b, sem_ob)
            compute(buf_b, acc_b)
            pltpu.async_copy(
                acc_b, out_hbm.at[pl.ds(tok0 + (c0 + 1) * _CH, _CH)], sem_ob)
            return carry

        lax.fori_loop(0, nchunk // 2, pbody, 0)
        wait_out(acc_a, sem_oa)
        wait_out(acc_b, sem_ob)

    return gather_sum


@functools.lru_cache(maxsize=None)
def _sc_gather_sum_cached(ntok):
    return _make_sc_gather_sum(ntok)


_CH2 = 4


def _make_sc_expand(ntok):
    tpw = ntok // _NW
    nchunk = tpw // _CH2
    rpc = _CH2 * _ND
    mesh = plsc.VectorSubcoreMesh(core_axis_name="c", subcore_axis_name="s")

    @functools.partial(
        pl.kernel,
        mesh=mesh,
        out_type=jax.ShapeDtypeStruct((ntok, _SA, _D), jnp.float32),
        scratch_types=[
            pltpu.VMEM((tpw * _ND,), jnp.int32),
            pltpu.VMEM((rpc, _D), jnp.float32),
            pltpu.VMEM((rpc, _D), jnp.float32),
            pltpu.VMEM((_CH2, _D), jnp.float32),
            pltpu.VMEM((_CH2, _D), jnp.float32),
            pltpu.VMEM((_SA, _D), jnp.float32),
            pltpu.VMEM((_CH2, _SA, _D), jnp.float32),
            pltpu.VMEM((_CH2, _SA, _D), jnp.float32),
            pltpu.SemaphoreType.DMA,
            pltpu.SemaphoreType.DMA,
            pltpu.SemaphoreType.DMA,
            pltpu.SemaphoreType.DMA,
            pltpu.SemaphoreType.DMA,
            pltpu.SemaphoreType.DMA,
        ],
    )
    def sc_expand(table_hbm, idx_hbm, cvec_hbm, base_hbm, out_hbm,
                  idx_v, gbuf_a, gbuf_b, cbuf_a, cbuf_b, base_v,
                  obuf_a, obuf_b, sem_ga, sem_gb, sem_ca, sem_cb,
                  sem_oa, sem_ob):
        wid = lax.axis_index("s") * _SC_CORES + lax.axis_index("c")
        tok0 = wid * tpw
        pltpu.sync_copy(idx_hbm.at[pl.ds(tok0 * _ND, tpw * _ND)], idx_v)
        pltpu.sync_copy(base_hbm, base_v)

        def start_in(c, gbuf, cbuf, sem_g, sem_c):
            pltpu.async_copy(
                table_hbm.at[idx_v.at[pl.ds(c * rpc, rpc)]], gbuf, sem_g)
            pltpu.async_copy(
                cvec_hbm.at[pl.ds(tok0 + c * _CH2, _CH2)], cbuf, sem_c)

        def wait_in(gbuf, cbuf, sem_g, sem_c):
            pltpu.make_async_copy(
                table_hbm.at[idx_v.at[pl.ds(0, rpc)]], gbuf, sem_g).wait()
            pltpu.make_async_copy(
                cvec_hbm.at[pl.ds(tok0, _CH2)], cbuf, sem_c).wait()

        def wait_out(obuf, sem):
            pltpu.make_async_copy(
                obuf, out_hbm.at[pl.ds(tok0, _CH2)], sem).wait()

        def compute(gbuf, cbuf, obuf):
            @plsc.parallel_loop(0, _D // 16, 1, unroll=2)
            def cbody(dd):
                sl = pl.ds(dd * 16, 16)
                for t in range(_CH2):
                    v = ((gbuf[4 * t + 0, sl] + gbuf[4 * t + 1, sl])
                         + (gbuf[4 * t + 2, sl] + gbuf[4 * t + 3, sl])
                         + cbuf[t, sl])
                    for s in range(_SA):
                        obuf[t, s, sl] = v + base_v[s, sl]

        start_in(0, gbuf_a, cbuf_a, sem_ga, sem_ca)

        def pbody(p, carry):
            c0 = 2 * p
            start_in(c0 + 1, gbuf_b, cbuf_b, sem_gb, sem_cb)
            wait_in(gbuf_a, cbuf_a, sem_ga, sem_ca)

            @pl.when(p > 0)
            def _():
                wait_out(obuf_a, sem_oa)
            compute(gbuf_a, cbuf_a, obuf_a)
            pltpu.async_copy(
                obuf_a, out_hbm.at[pl.ds(tok0 + c0 * _CH2, _CH2)], sem_oa)

            @pl.when(p + 1 < nchunk // 2)
            def _():
                start_in(c0 + 2, gbuf_a, cbuf_a, sem_ga, sem_ca)
            wait_in(gbuf_b, cbuf_b, sem_gb, sem_cb)

            @pl.when(p > 0)
            def _():
                wait_out(obuf_b, sem_ob)
            compute(gbuf_b, cbuf_b, obuf_b)
            pltpu.async_copy(
                obuf_b, out_hbm.at[pl.ds(tok0 + (c0 + 1) * _CH2, _CH2)],
                sem_ob)
            return carry

        lax.fori_loop(0, nchunk // 2, pbody, 0)
        wait_out(obuf_a, sem_oa)
        wait_out(obuf_b, sem_ob)

    return sc_expand


@functools.lru_cache(maxsize=2)
def _sc_expand_cached(ntok):
    return _make_sc_expand(ntok)


_PT = 256


def _pre_body(cont_ref, w_ref, comp_ref, lb_ref, cvec_ref, base_ref):
    cvec_ref[...] = jnp.dot(cont_ref[...], w_ref[...],
                            preferred_element_type=jnp.float32)
    base_ref[...] = (jnp.sum(comp_ref[...], axis=0)
                     + jnp.sum(lb_ref[...], axis=0)[None, :])


def _precompute(cont, w2d, comp, lin_b):
    return pl.pallas_call(
        _pre_body,
        grid=(_NTOK // _PT,),
        in_specs=[
            pl.BlockSpec((_PT, _NC), lambda i: (i, 0)),
            pl.BlockSpec((_NC, _D), lambda i: (0, 0)),
            pl.BlockSpec((_ND + _NC, _SA, _D), lambda i: (0, 0, 0)),
            pl.BlockSpec((_NC, _D), lambda i: (0, 0)),
        ],
        out_specs=[
            pl.BlockSpec((_PT, _D), lambda i: (i, 0)),
            pl.BlockSpec((_SA, _D), lambda i: (0, 0)),
        ],
        out_shape=[
            jax.ShapeDtypeStruct((_NTOK, _D), jnp.float32),
            jax.ShapeDtypeStruct((_SA, _D), jnp.float32),
        ],
        compiler_params=pltpu.CompilerParams(
            dimension_semantics=("arbitrary",)),
    )(cont, w2d, comp, lin_b)


_TT = 256


def _expand_first_body(vec_ref, cont_ref, w_ref, comp_ref, lb_ref, out_ref):
    base = jnp.sum(comp_ref[...], axis=0) + jnp.sum(lb_ref[...], axis=0)[None, :]
    tok = vec_ref[...] + jnp.dot(cont_ref[...], w_ref[...],
                                 preferred_element_type=jnp.float32)
    out_ref[...] = tok[:, None, :] + base[None, :, :]


def _expand_chain_body(prev_ref, vec_ref, cont_ref, w_ref, comp_ref, lb_ref,
                       out_ref):
    del prev_ref
    _expand_first_body(vec_ref, cont_ref, w_ref, comp_ref, lb_ref, out_ref)


def _expand_slice(tok_off, ntok, prev, vec, cont, w2d, comp, lin_b):
    nblk = ntok // _TT
    data_specs = [
        pl.BlockSpec((_TT, _D), lambda i: (i, 0)),
        pl.BlockSpec((_TT, _NC), lambda i: (i, 0)),
        pl.BlockSpec((_NC, _D), lambda i: (0, 0)),
        pl.BlockSpec((_ND + _NC, _SA, _D), lambda i: (0, 0, 0)),
        pl.BlockSpec((_NC, _D), lambda i: (0, 0)),
    ]
    blk0 = tok_off // _TT
    out_spec = pl.BlockSpec((_TT, _SA, _D),
                            lambda i, _b=blk0: (_b + i, 0, 0))
    out_shape = jax.ShapeDtypeStruct((_NTOK, _SA, _D), jnp.float32)
    params = pltpu.CompilerParams(dimension_semantics=("arbitrary",))
    if prev is None:
        return pl.pallas_call(
            _expand_first_body,
            grid=(nblk,),
            in_specs=data_specs,
            out_specs=out_spec,
            out_shape=out_shape,
            compiler_params=params,
        )(vec, cont, w2d, comp, lin_b)
    return pl.pallas_call(
        _expand_chain_body,
        grid=(nblk,),
        in_specs=[pl.BlockSpec(memory_space=pl.ANY)] + data_specs,
        out_specs=out_spec,
        out_shape=out_shape,
        input_output_aliases={0: 0},
        compiler_params=params,
    )(prev, vec, cont, w2d, comp, lin_b)


def kernel(discrete_actions, continuous_actions, emb_tables, lin_w, lin_b,
           component_tokens):
    table = emb_tables.reshape(_ND * _BINS, _D)
    idx = (discrete_actions.reshape(_NTOK, _ND).astype(jnp.int32)
           + (jnp.arange(_ND, dtype=jnp.int32) * _BINS)[None, :]).reshape(-1)
    cont = continuous_actions.reshape(_NTOK, _NC)
    w2d = lin_w[:, :, 0]
    comp = component_tokens.reshape(_ND + _NC, _SA, _D)

    slices = (1024, 1024, 1024, 1024)
    vecs = []
    off = 0
    for n in slices:
        vecs.append((off, n, _sc_gather_sum_cached(n)(
            table, idx[off * _ND:(off + n) * _ND])))
        off += n
    out = None
    for off, n, vec in vecs:
        out = _expand_slice(off, n, out, vec,
                            cont[off:off + n], w2d, comp, lin_b)
    return out.reshape(_B, _T, _SA, _D)


# speedup vs baseline: 1.1934x; 1.1934x over previous
import functools

import jax
import jax.numpy as jnp
from jax import lax
from jax.experimental import pallas as pl
from jax.experimental.pallas import tpu as pltpu
from jax.experimental.pallas import tpu_sc as plsc

_B = 16
_T = 256
_ND = 4
_NC = 6
_BINS = 256
_SA = 8
_D = 1024
_NTOK = _B * _T

_NSLICE = 4
_SLICE = _NTOK // _NSLICE

_SC_CORES = 2
_SC_SUBCORES = 16
_NW = _SC_CORES * _SC_SUBCORES
_CH = 8
_RPC = _CH * _ND


def _make_sc_gather_sum(ntok):
    tpw = ntok // _NW
    nchunk = tpw // _CH
    mesh = plsc.VectorSubcoreMesh(core_axis_name="c", subcore_axis_name="s")

    @functools.partial(
        pl.kernel,
        mesh=mesh,
        out_type=jax.ShapeDtypeStruct((ntok, _D), jnp.float32),
        scratch_types=[
            pltpu.VMEM((tpw * _ND,), jnp.int32),
            pltpu.VMEM((_RPC, _D), jnp.float32),
            pltpu.VMEM((_RPC, _D), jnp.float32),
            pltpu.VMEM((_CH, _D), jnp.float32),
            pltpu.VMEM((_CH, _D), jnp.float32),
            pltpu.SemaphoreType.DMA,
            pltpu.SemaphoreType.DMA,
            pltpu.SemaphoreType.DMA,
            pltpu.SemaphoreType.DMA,
        ],
    )
    def gather_sum(table_hbm, idx_hbm, out_hbm, idx_v, buf_a, buf_b,
                   acc_a, acc_b, sem_a, sem_b, sem_oa, sem_ob):
        wid = lax.axis_index("s") * _SC_CORES + lax.axis_index("c")
        tok0 = wid * tpw
        pltpu.sync_copy(idx_hbm.at[pl.ds(tok0 * _ND, tpw * _ND)], idx_v)

        def compute(buf, acc):
            @plsc.parallel_loop(0, _CH * 16, 1, unroll=4)
            def cbody(i):
                t = i >> 4
                dd = i & 15
                for u in range(4):
                    sl = pl.ds(dd * 64 + u * 16, 16)
                    acc[t, sl] = ((buf[4 * t + 0, sl] + buf[4 * t + 1, sl])
                                  + (buf[4 * t + 2, sl] + buf[4 * t + 3, sl]))

        def wait_gather(buf, sem):
            pltpu.make_async_copy(
                table_hbm.at[idx_v.at[pl.ds(0, _RPC)]], buf, sem).wait()

        def wait_out(acc, sem):
            pltpu.make_async_copy(
                acc, out_hbm.at[pl.ds(tok0, _CH)], sem).wait()

        pltpu.async_copy(table_hbm.at[idx_v.at[pl.ds(0, _RPC)]], buf_a, sem_a)

        def pbody(p, carry):
            c0 = 2 * p
            pltpu.async_copy(
                table_hbm.at[idx_v.at[pl.ds((c0 + 1) * _RPC, _RPC)]],
                buf_b, sem_b)
            wait_gather(buf_a, sem_a)

            @pl.when(p > 0)
            def _():
                wait_out(acc_a, sem_oa)
            compute(buf_a, acc_a)
            pltpu.async_copy(
                acc_a, out_hbm.at[pl.ds(tok0 + c0 * _CH, _CH)], sem_oa)

            @pl.when(p + 1 < nchunk // 2)
            def _():
                pltpu.async_copy(
                    table_hbm.at[idx_v.at[pl.ds((c0 + 2) * _RPC, _RPC)]],
                    buf_a, sem_a)
            wait_gather(buf_b, sem_b)

            @pl.when(p > 0)
            def _():
                wait_out(acc_b, sem_ob)
            compute(buf_b, acc_b)
            pltpu.async_copy(
                acc_b, out_hbm.at[pl.ds(tok0 + (c0 + 1) * _CH, _CH)], sem_ob)
            return carry

        lax.fori_loop(0, nchunk // 2, pbody, 0)
        wait_out(acc_a, sem_oa)
        wait_out(acc_b, sem_ob)

    return gather_sum


@functools.lru_cache(maxsize=None)
def _sc_gather_sum_cached(ntok):
    return _make_sc_gather_sum(ntok)


_CH2 = 4


def _make_sc_expand(ntok):
    tpw = ntok // _NW
    nchunk = tpw // _CH2
    rpc = _CH2 * _ND
    mesh = plsc.VectorSubcoreMesh(core_axis_name="c", subcore_axis_name="s")

    @functools.partial(
        pl.kernel,
        mesh=mesh,
        out_type=jax.ShapeDtypeStruct((ntok, _SA, _D), jnp.float32),
        scratch_types=[
            pltpu.VMEM((tpw * _ND,), jnp.int32),
            pltpu.VMEM((rpc, _D), jnp.float32),
            pltpu.VMEM((rpc, _D), jnp.float32),
            pltpu.VMEM((_CH2, _D), jnp.float32),
            pltpu.VMEM((_CH2, _D), jnp.float32),
            pltpu.VMEM((_SA, _D), jnp.float32),
            pltpu.VMEM((_CH2, _SA, _D), jnp.float32),
            pltpu.VMEM((_CH2, _SA, _D), jnp.float32),
            pltpu.SemaphoreType.DMA,
            pltpu.SemaphoreType.DMA,
            pltpu.SemaphoreType.DMA,
            pltpu.SemaphoreType.DMA,
            pltpu.SemaphoreType.DMA,
            pltpu.SemaphoreType.DMA,
        ],
    )
    def sc_expand(table_hbm, idx_hbm, cvec_hbm, base_hbm, out_hbm,
                  idx_v, gbuf_a, gbuf_b, cbuf_a, cbuf_b, base_v,
                  obuf_a, obuf_b, sem_ga, sem_gb, sem_ca, sem_cb,
                  sem_oa, sem_ob):
        wid = lax.axis_index("s") * _SC_CORES + lax.axis_index("c")
        tok0 = wid * tpw
        pltpu.sync_copy(idx_hbm.at[pl.ds(tok0 * _ND, tpw * _ND)], idx_v)
        pltpu.sync_copy(base_hbm, base_v)

        def start_in(c, gbuf, cbuf, sem_g, sem_c):
            pltpu.async_copy(
                table_hbm.at[idx_v.at[pl.ds(c * rpc, rpc)]], gbuf, sem_g)
            pltpu.async_copy(
                cvec_hbm.at[pl.ds(tok0 + c * _CH2, _CH2)], cbuf, sem_c)

        def wait_in(gbuf, cbuf, sem_g, sem_c):
            pltpu.make_async_copy(
                table_hbm.at[idx_v.at[pl.ds(0, rpc)]], gbuf, sem_g).wait()
            pltpu.make_async_copy(
                cvec_hbm.at[pl.ds(tok0, _CH2)], cbuf, sem_c).wait()

        def wait_out(obuf, sem):
            pltpu.make_async_copy(
                obuf, out_hbm.at[pl.ds(tok0, _CH2)], sem).wait()

        def compute(gbuf, cbuf, obuf):
            @plsc.parallel_loop(0, _D // 16, 1, unroll=2)
            def cbody(dd):
                sl = pl.ds(dd * 16, 16)
                for t in range(_CH2):
                    v = ((gbuf[4 * t + 0, sl] + gbuf[4 * t + 1, sl])
                         + (gbuf[4 * t + 2, sl] + gbuf[4 * t + 3, sl])
                         + cbuf[t, sl])
                    for s in range(_SA):
                        obuf[t, s, sl] = v + base_v[s, sl]

        start_in(0, gbuf_a, cbuf_a, sem_ga, sem_ca)

        def pbody(p, carry):
            c0 = 2 * p
            start_in(c0 + 1, gbuf_b, cbuf_b, sem_gb, sem_cb)
            wait_in(gbuf_a, cbuf_a, sem_ga, sem_ca)

            @pl.when(p > 0)
            def _():
                wait_out(obuf_a, sem_oa)
            compute(gbuf_a, cbuf_a, obuf_a)
            pltpu.async_copy(
                obuf_a, out_hbm.at[pl.ds(tok0 + c0 * _CH2, _CH2)], sem_oa)

            @pl.when(p + 1 < nchunk // 2)
            def _():
                start_in(c0 + 2, gbuf_a, cbuf_a, sem_ga, sem_ca)
            wait_in(gbuf_b, cbuf_b, sem_gb, sem_cb)

            @pl.when(p > 0)
            def _():
                wait_out(obuf_b, sem_ob)
            compute(gbuf_b, cbuf_b, obuf_b)
            pltpu.async_copy(
                obuf_b, out_hbm.at[pl.ds(tok0 + (c0 + 1) * _CH2, _CH2)],
                sem_ob)
            return carry

        lax.fori_loop(0, nchunk // 2, pbody, 0)
        wait_out(obuf_a, sem_oa)
        wait_out(obuf_b, sem_ob)

    return sc_expand


@functools.lru_cache(maxsize=2)
def _sc_expand_cached(ntok):
    return _make_sc_expand(ntok)


_PT = 256


def _pre_body(cont_ref, w_ref, comp_ref, lb_ref, cvec_ref, base_ref):
    cvec_ref[...] = jnp.dot(cont_ref[...], w_ref[...],
                            preferred_element_type=jnp.float32)
    base_ref[...] = (jnp.sum(comp_ref[...], axis=0)
                     + jnp.sum(lb_ref[...], axis=0)[None, :])


def _precompute(cont, w2d, comp, lin_b):
    return pl.pallas_call(
        _pre_body,
        grid=(_NTOK // _PT,),
        in_specs=[
            pl.BlockSpec((_PT, _NC), lambda i: (i, 0)),
            pl.BlockSpec((_NC, _D), lambda i: (0, 0)),
            pl.BlockSpec((_ND + _NC, _SA, _D), lambda i: (0, 0, 0)),
            pl.BlockSpec((_NC, _D), lambda i: (0, 0)),
        ],
        out_specs=[
            pl.BlockSpec((_PT, _D), lambda i: (i, 0)),
            pl.BlockSpec((_SA, _D), lambda i: (0, 0)),
        ],
        out_shape=[
            jax.ShapeDtypeStruct((_NTOK, _D), jnp.float32),
            jax.ShapeDtypeStruct((_SA, _D), jnp.float32),
        ],
        compiler_params=pltpu.CompilerParams(
            dimension_semantics=("arbitrary",)),
    )(cont, w2d, comp, lin_b)


_TT = 256


def _expand_first_body(vec_ref, cont_ref, w_ref, comp_ref, lb_ref, out_ref):
    base = jnp.sum(comp_ref[...], axis=0) + jnp.sum(lb_ref[...], axis=0)[None, :]
    tok = vec_ref[...] + jnp.dot(cont_ref[...], w_ref[...],
                                 preferred_element_type=jnp.float32)
    out_ref[...] = tok[:, None, :] + base[None, :, :]


def _expand_chain_body(prev_ref, vec_ref, cont_ref, w_ref, comp_ref, lb_ref,
                       out_ref):
    del prev_ref
    _expand_first_body(vec_ref, cont_ref, w_ref, comp_ref, lb_ref, out_ref)


def _expand_onehot_body(disc_ref, tbl_ref, cont_ref, w_ref, comp_ref, lb_ref,
                        out_ref):
    tt = disc_ref.shape[0]
    iota = lax.broadcasted_iota(jnp.int32, (tt, _ND, _BINS), 2)
    oh = (iota == disc_ref[...][:, :, None]).astype(jnp.bfloat16)
    vec = jnp.dot(oh.reshape(tt, _ND * _BINS), tbl_ref[...],
                  preferred_element_type=jnp.float32)
    base = jnp.sum(comp_ref[...], axis=0) + jnp.sum(lb_ref[...], axis=0)[None, :]
    tok = vec + jnp.dot(cont_ref[...], w_ref[...],
                        preferred_element_type=jnp.float32)
    out_ref[...] = tok[:, None, :] + base[None, :, :]


def _expand_slice0_onehot(ntok, disc, tbl_bf, cont, w2d, comp, lin_b):
    nblk = ntok // _TT
    return pl.pallas_call(
        _expand_onehot_body,
        grid=(nblk,),
        in_specs=[
            pl.BlockSpec((_TT, _ND), lambda i: (i, 0)),
            pl.BlockSpec((_ND * _BINS, _D), lambda i: (0, 0)),
            pl.BlockSpec((_TT, _NC), lambda i: (i, 0)),
            pl.BlockSpec((_NC, _D), lambda i: (0, 0)),
            pl.BlockSpec((_ND + _NC, _SA, _D), lambda i: (0, 0, 0)),
            pl.BlockSpec((_NC, _D), lambda i: (0, 0)),
        ],
        out_specs=pl.BlockSpec((_TT, _SA, _D), lambda i: (i, 0, 0)),
        out_shape=jax.ShapeDtypeStruct((_NTOK, _SA, _D), jnp.float32),
        compiler_params=pltpu.CompilerParams(
            dimension_semantics=("arbitrary",)),
    )(disc, tbl_bf, cont, w2d, comp, lin_b)


def _expand_slice(tok_off, ntok, prev, vec, cont, w2d, comp, lin_b):
    nblk = ntok // _TT
    data_specs = [
        pl.BlockSpec((_TT, _D), lambda i: (i, 0)),
        pl.BlockSpec((_TT, _NC), lambda i: (i, 0)),
        pl.BlockSpec((_NC, _D), lambda i: (0, 0)),
        pl.BlockSpec((_ND + _NC, _SA, _D), lambda i: (0, 0, 0)),
        pl.BlockSpec((_NC, _D), lambda i: (0, 0)),
    ]
    blk0 = tok_off // _TT
    out_spec = pl.BlockSpec((_TT, _SA, _D),
                            lambda i, _b=blk0: (_b + i, 0, 0))
    out_shape = jax.ShapeDtypeStruct((_NTOK, _SA, _D), jnp.float32)
    params = pltpu.CompilerParams(dimension_semantics=("arbitrary",))
    if prev is None:
        return pl.pallas_call(
            _expand_first_body,
            grid=(nblk,),
            in_specs=data_specs,
            out_specs=out_spec,
            out_shape=out_shape,
            compiler_params=params,
        )(vec, cont, w2d, comp, lin_b)
    return pl.pallas_call(
        _expand_chain_body,
        grid=(nblk,),
        in_specs=[pl.BlockSpec(memory_space=pl.ANY)] + data_specs,
        out_specs=out_spec,
        out_shape=out_shape,
        input_output_aliases={0: 0},
        compiler_params=params,
    )(prev, vec, cont, w2d, comp, lin_b)


def kernel(discrete_actions, continuous_actions, emb_tables, lin_w, lin_b,
           component_tokens):
    table = emb_tables.reshape(_ND * _BINS, _D)
    idx = (discrete_actions.reshape(_NTOK, _ND).astype(jnp.int32)
           + (jnp.arange(_ND, dtype=jnp.int32) * _BINS)[None, :]).reshape(-1)
    cont = continuous_actions.reshape(_NTOK, _NC)
    w2d = lin_w[:, :, 0]
    comp = component_tokens.reshape(_ND + _NC, _SA, _D)

    slices = (1024, 1024, 1024, 1024)
    disc2d = discrete_actions.reshape(_NTOK, _ND).astype(jnp.int32)
    tbl_bf = table.astype(jnp.bfloat16)
    vecs = []
    off = slices[0]
    for n in slices[1:]:
        vecs.append((off, n, _sc_gather_sum_cached(n)(
            table, idx[off * _ND:(off + n) * _ND])))
        off += n
    out = _expand_slice0_onehot(slices[0], disc2d[:slices[0]], tbl_bf,
                                cont[:slices[0]], w2d, comp, lin_b)
    for off, n, vec in vecs:
        out = _expand_slice(off, n, out, vec,
                            cont[off:off + n], w2d, comp, lin_b)
    return out.reshape(_B, _T, _SA, _D)


# speedup vs baseline: 1.2838x; 1.0758x over previous
import functools

import jax
import jax.numpy as jnp
from jax import lax
from jax.experimental import pallas as pl
from jax.experimental.pallas import tpu as pltpu
from jax.experimental.pallas import tpu_sc as plsc

_B = 16
_T = 256
_ND = 4
_NC = 6
_BINS = 256
_SA = 8
_D = 1024
_NTOK = _B * _T

_NSLICE = 4
_SLICE = _NTOK // _NSLICE

_SC_CORES = 2
_SC_SUBCORES = 16
_NW = _SC_CORES * _SC_SUBCORES
_CH = 8
_RPC = _CH * _ND


def _make_sc_gather_sum(ntok):
    tpw = ntok // _NW
    nchunk = tpw // _CH
    mesh = plsc.VectorSubcoreMesh(core_axis_name="c", subcore_axis_name="s")

    @functools.partial(
        pl.kernel,
        mesh=mesh,
        out_type=jax.ShapeDtypeStruct((ntok, _D), jnp.float32),
        scratch_types=[
            pltpu.VMEM((tpw * _ND,), jnp.int32),
            pltpu.VMEM((_RPC, _D), jnp.float32),
            pltpu.VMEM((_RPC, _D), jnp.float32),
            pltpu.VMEM((_CH, _D), jnp.float32),
            pltpu.VMEM((_CH, _D), jnp.float32),
            pltpu.SemaphoreType.DMA,
            pltpu.SemaphoreType.DMA,
            pltpu.SemaphoreType.DMA,
            pltpu.SemaphoreType.DMA,
        ],
    )
    def gather_sum(table_hbm, idx_hbm, out_hbm, idx_v, buf_a, buf_b,
                   acc_a, acc_b, sem_a, sem_b, sem_oa, sem_ob):
        wid = lax.axis_index("s") * _SC_CORES + lax.axis_index("c")
        tok0 = wid * tpw
        pltpu.sync_copy(idx_hbm.at[pl.ds(tok0 * _ND, tpw * _ND)], idx_v)

        def compute(buf, acc):
            @plsc.parallel_loop(0, _CH * 16, 1, unroll=4)
            def cbody(i):
                t = i >> 4
                dd = i & 15
                for u in range(4):
                    sl = pl.ds(dd * 64 + u * 16, 16)
                    acc[t, sl] = ((buf[4 * t + 0, sl] + buf[4 * t + 1, sl])
                                  + (buf[4 * t + 2, sl] + buf[4 * t + 3, sl]))

        def wait_gather(buf, sem):
            pltpu.make_async_copy(
                table_hbm.at[idx_v.at[pl.ds(0, _RPC)]], buf, sem).wait()

        def wait_out(acc, sem):
            pltpu.make_async_copy(
                acc, out_hbm.at[pl.ds(tok0, _CH)], sem).wait()

        pltpu.async_copy(table_hbm.at[idx_v.at[pl.ds(0, _RPC)]], buf_a, sem_a)

        def pbody(p, carry):
            c0 = 2 * p
            pltpu.async_copy(
                table_hbm.at[idx_v.at[pl.ds((c0 + 1) * _RPC, _RPC)]],
                buf_b, sem_b)
            wait_gather(buf_a, sem_a)

            @pl.when(p > 0)
            def _():
                wait_out(acc_a, sem_oa)
            compute(buf_a, acc_a)
            pltpu.async_copy(
                acc_a, out_hbm.at[pl.ds(tok0 + c0 * _CH, _CH)], sem_oa)

            @pl.when(p + 1 < nchunk // 2)
            def _():
                pltpu.async_copy(
                    table_hbm.at[idx_v.at[pl.ds((c0 + 2) * _RPC, _RPC)]],
                    buf_a, sem_a)
            wait_gather(buf_b, sem_b)

            @pl.when(p > 0)
            def _():
                wait_out(acc_b, sem_ob)
            compute(buf_b, acc_b)
            pltpu.async_copy(
                acc_b, out_hbm.at[pl.ds(tok0 + (c0 + 1) * _CH, _CH)], sem_ob)
            return carry

        lax.fori_loop(0, nchunk // 2, pbody, 0)
        wait_out(acc_a, sem_oa)
        wait_out(acc_b, sem_ob)

    return gather_sum


@functools.lru_cache(maxsize=None)
def _sc_gather_sum_cached(ntok):
    return _make_sc_gather_sum(ntok)


_CH2 = 4


def _make_sc_expand(ntok):
    tpw = ntok // _NW
    nchunk = tpw // _CH2
    rpc = _CH2 * _ND
    mesh = plsc.VectorSubcoreMesh(core_axis_name="c", subcore_axis_name="s")

    @functools.partial(
        pl.kernel,
        mesh=mesh,
        out_type=jax.ShapeDtypeStruct((ntok, _SA, _D), jnp.float32),
        scratch_types=[
            pltpu.VMEM((tpw * _ND,), jnp.int32),
            pltpu.VMEM((rpc, _D), jnp.float32),
            pltpu.VMEM((rpc, _D), jnp.float32),
            pltpu.VMEM((_CH2, _D), jnp.float32),
            pltpu.VMEM((_CH2, _D), jnp.float32),
            pltpu.VMEM((_SA, _D), jnp.float32),
            pltpu.VMEM((_CH2, _SA, _D), jnp.float32),
            pltpu.VMEM((_CH2, _SA, _D), jnp.float32),
            pltpu.SemaphoreType.DMA,
            pltpu.SemaphoreType.DMA,
            pltpu.SemaphoreType.DMA,
            pltpu.SemaphoreType.DMA,
            pltpu.SemaphoreType.DMA,
            pltpu.SemaphoreType.DMA,
        ],
    )
    def sc_expand(table_hbm, idx_hbm, cvec_hbm, base_hbm, out_hbm,
                  idx_v, gbuf_a, gbuf_b, cbuf_a, cbuf_b, base_v,
                  obuf_a, obuf_b, sem_ga, sem_gb, sem_ca, sem_cb,
                  sem_oa, sem_ob):
        wid = lax.axis_index("s") * _SC_CORES + lax.axis_index("c")
        tok0 = wid * tpw
        pltpu.sync_copy(idx_hbm.at[pl.ds(tok0 * _ND, tpw * _ND)], idx_v)
        pltpu.sync_copy(base_hbm, base_v)

        def start_in(c, gbuf, cbuf, sem_g, sem_c):
            pltpu.async_copy(
                table_hbm.at[idx_v.at[pl.ds(c * rpc, rpc)]], gbuf, sem_g)
            pltpu.async_copy(
                cvec_hbm.at[pl.ds(tok0 + c * _CH2, _CH2)], cbuf, sem_c)

        def wait_in(gbuf, cbuf, sem_g, sem_c):
            pltpu.make_async_copy(
                table_hbm.at[idx_v.at[pl.ds(0, rpc)]], gbuf, sem_g).wait()
            pltpu.make_async_copy(
                cvec_hbm.at[pl.ds(tok0, _CH2)], cbuf, sem_c).wait()

        def wait_out(obuf, sem):
            pltpu.make_async_copy(
                obuf, out_hbm.at[pl.ds(tok0, _CH2)], sem).wait()

        def compute(gbuf, cbuf, obuf):
            @plsc.parallel_loop(0, _D // 16, 1, unroll=2)
            def cbody(dd):
                sl = pl.ds(dd * 16, 16)
                for t in range(_CH2):
                    v = ((gbuf[4 * t + 0, sl] + gbuf[4 * t + 1, sl])
                         + (gbuf[4 * t + 2, sl] + gbuf[4 * t + 3, sl])
                         + cbuf[t, sl])
                    for s in range(_SA):
                        obuf[t, s, sl] = v + base_v[s, sl]

        start_in(0, gbuf_a, cbuf_a, sem_ga, sem_ca)

        def pbody(p, carry):
            c0 = 2 * p
            start_in(c0 + 1, gbuf_b, cbuf_b, sem_gb, sem_cb)
            wait_in(gbuf_a, cbuf_a, sem_ga, sem_ca)

            @pl.when(p > 0)
            def _():
                wait_out(obuf_a, sem_oa)
            compute(gbuf_a, cbuf_a, obuf_a)
            pltpu.async_copy(
                obuf_a, out_hbm.at[pl.ds(tok0 + c0 * _CH2, _CH2)], sem_oa)

            @pl.when(p + 1 < nchunk // 2)
            def _():
                start_in(c0 + 2, gbuf_a, cbuf_a, sem_ga, sem_ca)
            wait_in(gbuf_b, cbuf_b, sem_gb, sem_cb)

            @pl.when(p > 0)
            def _():
                wait_out(obuf_b, sem_ob)
            compute(gbuf_b, cbuf_b, obuf_b)
            pltpu.async_copy(
                obuf_b, out_hbm.at[pl.ds(tok0 + (c0 + 1) * _CH2, _CH2)],
                sem_ob)
            return carry

        lax.fori_loop(0, nchunk // 2, pbody, 0)
        wait_out(obuf_a, sem_oa)
        wait_out(obuf_b, sem_ob)

    return sc_expand


@functools.lru_cache(maxsize=2)
def _sc_expand_cached(ntok):
    return _make_sc_expand(ntok)


_PT = 256


def _pre_body(cont_ref, w_ref, comp_ref, lb_ref, cvec_ref, base_ref):
    cvec_ref[...] = jnp.dot(cont_ref[...], w_ref[...],
                            preferred_element_type=jnp.float32)
    base_ref[...] = (jnp.sum(comp_ref[...], axis=0)
                     + jnp.sum(lb_ref[...], axis=0)[None, :])


def _precompute(cont, w2d, comp, lin_b):
    return pl.pallas_call(
        _pre_body,
        grid=(_NTOK // _PT,),
        in_specs=[
            pl.BlockSpec((_PT, _NC), lambda i: (i, 0)),
            pl.BlockSpec((_NC, _D), lambda i: (0, 0)),
            pl.BlockSpec((_ND + _NC, _SA, _D), lambda i: (0, 0, 0)),
            pl.BlockSpec((_NC, _D), lambda i: (0, 0)),
        ],
        out_specs=[
            pl.BlockSpec((_PT, _D), lambda i: (i, 0)),
            pl.BlockSpec((_SA, _D), lambda i: (0, 0)),
        ],
        out_shape=[
            jax.ShapeDtypeStruct((_NTOK, _D), jnp.float32),
            jax.ShapeDtypeStruct((_SA, _D), jnp.float32),
        ],
        compiler_params=pltpu.CompilerParams(
            dimension_semantics=("arbitrary",)),
    )(cont, w2d, comp, lin_b)


_TT = 256


def _expand_first_body(vec_ref, cont_ref, w_ref, comp_ref, lb_ref, out_ref):
    base = jnp.sum(comp_ref[...], axis=0) + jnp.sum(lb_ref[...], axis=0)[None, :]
    tok = vec_ref[...] + jnp.dot(cont_ref[...], w_ref[...],
                                 preferred_element_type=jnp.float32)
    out_ref[...] = tok[:, None, :] + base[None, :, :]


def _expand_chain_body(prev_ref, vec_ref, cont_ref, w_ref, comp_ref, lb_ref,
                       out_ref):
    del prev_ref
    _expand_first_body(vec_ref, cont_ref, w_ref, comp_ref, lb_ref, out_ref)


def _expand_onehot_body(disc_ref, tbl_ref, cont_ref, w_ref, comp_ref, lb_ref,
                        out_ref):
    tt = disc_ref.shape[0]
    iota = lax.broadcasted_iota(jnp.int32, (tt, _ND, _BINS), 2)
    oh = (iota == disc_ref[...][:, :, None]).astype(jnp.bfloat16)
    vec = jnp.dot(oh.reshape(tt, _ND * _BINS), tbl_ref[...],
                  preferred_element_type=jnp.float32)
    base = jnp.sum(comp_ref[...], axis=0) + jnp.sum(lb_ref[...], axis=0)[None, :]
    tok = vec + jnp.dot(cont_ref[...], w_ref[...],
                        preferred_element_type=jnp.float32)
    out_ref[...] = tok[:, None, :] + base[None, :, :]


def _expand_slice0_onehot(ntok, disc, tbl_bf, cont, w2d, comp, lin_b):
    nblk = ntok // _TT
    return pl.pallas_call(
        _expand_onehot_body,
        grid=(nblk,),
        in_specs=[
            pl.BlockSpec((_TT, _ND), lambda i: (i, 0)),
            pl.BlockSpec((_ND * _BINS, _D), lambda i: (0, 0)),
            pl.BlockSpec((_TT, _NC), lambda i: (i, 0)),
            pl.BlockSpec((_NC, _D), lambda i: (0, 0)),
            pl.BlockSpec((_ND + _NC, _SA, _D), lambda i: (0, 0, 0)),
            pl.BlockSpec((_NC, _D), lambda i: (0, 0)),
        ],
        out_specs=pl.BlockSpec((_TT, _SA, _D), lambda i: (i, 0, 0)),
        out_shape=jax.ShapeDtypeStruct((_NTOK, _SA, _D), jnp.float32),
        compiler_params=pltpu.CompilerParams(
            dimension_semantics=("arbitrary",)),
    )(disc, tbl_bf, cont, w2d, comp, lin_b)


def _expand_slice(tok_off, ntok, prev, vec, cont, w2d, comp, lin_b):
    nblk = ntok // _TT
    data_specs = [
        pl.BlockSpec((_TT, _D), lambda i: (i, 0)),
        pl.BlockSpec((_TT, _NC), lambda i: (i, 0)),
        pl.BlockSpec((_NC, _D), lambda i: (0, 0)),
        pl.BlockSpec((_ND + _NC, _SA, _D), lambda i: (0, 0, 0)),
        pl.BlockSpec((_NC, _D), lambda i: (0, 0)),
    ]
    blk0 = tok_off // _TT
    out_spec = pl.BlockSpec((_TT, _SA, _D),
                            lambda i, _b=blk0: (_b + i, 0, 0))
    out_shape = jax.ShapeDtypeStruct((_NTOK, _SA, _D), jnp.float32)
    params = pltpu.CompilerParams(dimension_semantics=("arbitrary",))
    if prev is None:
        return pl.pallas_call(
            _expand_first_body,
            grid=(nblk,),
            in_specs=data_specs,
            out_specs=out_spec,
            out_shape=out_shape,
            compiler_params=params,
        )(vec, cont, w2d, comp, lin_b)
    return pl.pallas_call(
        _expand_chain_body,
        grid=(nblk,),
        in_specs=[pl.BlockSpec(memory_space=pl.ANY)] + data_specs,
        out_specs=out_spec,
        out_shape=out_shape,
        input_output_aliases={0: 0},
        compiler_params=params,
    )(prev, vec, cont, w2d, comp, lin_b)


def kernel(discrete_actions, continuous_actions, emb_tables, lin_w, lin_b,
           component_tokens):
    table = emb_tables.reshape(_ND * _BINS, _D)
    idx = (discrete_actions.reshape(_NTOK, _ND).astype(jnp.int32)
           + (jnp.arange(_ND, dtype=jnp.int32) * _BINS)[None, :]).reshape(-1)
    cont = continuous_actions.reshape(_NTOK, _NC)
    w2d = lin_w[:, :, 0]
    comp = component_tokens.reshape(_ND + _NC, _SA, _D)

    slices = (2048, 1024, 1024)
    disc2d = discrete_actions.reshape(_NTOK, _ND).astype(jnp.int32)
    tbl_bf = table.astype(jnp.bfloat16)
    vecs = []
    off = slices[0]
    for n in slices[1:]:
        vecs.append((off, n, _sc_gather_sum_cached(n)(
            table, idx[off * _ND:(off + n) * _ND])))
        off += n
    out = _expand_slice0_onehot(slices[0], disc2d[:slices[0]], tbl_bf,
                                cont[:slices[0]], w2d, comp, lin_b)
    for off, n, vec in vecs:
        out = _expand_slice(off, n, out, vec,
                            cont[off:off + n], w2d, comp, lin_b)
    return out.reshape(_B, _T, _SA, _D)


# speedup vs baseline: 1.3853x; 1.0790x over previous
import functools

import jax
import jax.numpy as jnp
from jax import lax
from jax.experimental import pallas as pl
from jax.experimental.pallas import tpu as pltpu
from jax.experimental.pallas import tpu_sc as plsc

_B = 16
_T = 256
_ND = 4
_NC = 6
_BINS = 256
_SA = 8
_D = 1024
_NTOK = _B * _T

_NSLICE = 4
_SLICE = _NTOK // _NSLICE

_SC_CORES = 2
_SC_SUBCORES = 16
_NW = _SC_CORES * _SC_SUBCORES
_CH = 8
_RPC = _CH * _ND


def _make_sc_gather_sum(ntok):
    tpw = ntok // _NW
    nchunk = tpw // _CH
    mesh = plsc.VectorSubcoreMesh(core_axis_name="c", subcore_axis_name="s")

    @functools.partial(
        pl.kernel,
        mesh=mesh,
        out_type=jax.ShapeDtypeStruct((ntok, _D), jnp.float32),
        scratch_types=[
            pltpu.VMEM((tpw * _ND,), jnp.int32),
            pltpu.VMEM((_RPC, _D), jnp.float32),
            pltpu.VMEM((_RPC, _D), jnp.float32),
            pltpu.VMEM((_CH, _D), jnp.float32),
            pltpu.VMEM((_CH, _D), jnp.float32),
            pltpu.SemaphoreType.DMA,
            pltpu.SemaphoreType.DMA,
            pltpu.SemaphoreType.DMA,
            pltpu.SemaphoreType.DMA,
        ],
    )
    def gather_sum(table_hbm, idx_hbm, out_hbm, idx_v, buf_a, buf_b,
                   acc_a, acc_b, sem_a, sem_b, sem_oa, sem_ob):
        wid = lax.axis_index("s") * _SC_CORES + lax.axis_index("c")
        tok0 = wid * tpw
        pltpu.sync_copy(idx_hbm.at[pl.ds(tok0 * _ND, tpw * _ND)], idx_v)

        def compute(buf, acc):
            @plsc.parallel_loop(0, _CH * 16, 1, unroll=4)
            def cbody(i):
                t = i >> 4
                dd = i & 15
                for u in range(4):
                    sl = pl.ds(dd * 64 + u * 16, 16)
                    acc[t, sl] = ((buf[4 * t + 0, sl] + buf[4 * t + 1, sl])
                                  + (buf[4 * t + 2, sl] + buf[4 * t + 3, sl]))

        def wait_gather(buf, sem):
            pltpu.make_async_copy(
                table_hbm.at[idx_v.at[pl.ds(0, _RPC)]], buf, sem).wait()

        def wait_out(acc, sem):
            pltpu.make_async_copy(
                acc, out_hbm.at[pl.ds(tok0, _CH)], sem).wait()

        pltpu.async_copy(table_hbm.at[idx_v.at[pl.ds(0, _RPC)]], buf_a, sem_a)

        def pbody(p, carry):
            c0 = 2 * p
            pltpu.async_copy(
                table_hbm.at[idx_v.at[pl.ds((c0 + 1) * _RPC, _RPC)]],
                buf_b, sem_b)
            wait_gather(buf_a, sem_a)

            @pl.when(p > 0)
            def _():
                wait_out(acc_a, sem_oa)
            compute(buf_a, acc_a)
            pltpu.async_copy(
                acc_a, out_hbm.at[pl.ds(tok0 + c0 * _CH, _CH)], sem_oa)

            @pl.when(p + 1 < nchunk // 2)
            def _():
                pltpu.async_copy(
                    table_hbm.at[idx_v.at[pl.ds((c0 + 2) * _RPC, _RPC)]],
                    buf_a, sem_a)
            wait_gather(buf_b, sem_b)

            @pl.when(p > 0)
            def _():
                wait_out(acc_b, sem_ob)
            compute(buf_b, acc_b)
            pltpu.async_copy(
                acc_b, out_hbm.at[pl.ds(tok0 + (c0 + 1) * _CH, _CH)], sem_ob)
            return carry

        lax.fori_loop(0, nchunk // 2, pbody, 0)
        wait_out(acc_a, sem_oa)
        wait_out(acc_b, sem_ob)

    return gather_sum


@functools.lru_cache(maxsize=None)
def _sc_gather_sum_cached(ntok):
    return _make_sc_gather_sum(ntok)


_CH2 = 4


def _make_sc_expand(ntok):
    tpw = ntok // _NW
    nchunk = tpw // _CH2
    rpc = _CH2 * _ND
    mesh = plsc.VectorSubcoreMesh(core_axis_name="c", subcore_axis_name="s")

    @functools.partial(
        pl.kernel,
        mesh=mesh,
        out_type=jax.ShapeDtypeStruct((ntok, _SA, _D), jnp.float32),
        scratch_types=[
            pltpu.VMEM((tpw * _ND,), jnp.int32),
            pltpu.VMEM((rpc, _D), jnp.float32),
            pltpu.VMEM((rpc, _D), jnp.float32),
            pltpu.VMEM((_CH2, _D), jnp.float32),
            pltpu.VMEM((_CH2, _D), jnp.float32),
            pltpu.VMEM((_SA, _D), jnp.float32),
            pltpu.VMEM((_CH2, _SA, _D), jnp.float32),
            pltpu.VMEM((_CH2, _SA, _D), jnp.float32),
            pltpu.SemaphoreType.DMA,
            pltpu.SemaphoreType.DMA,
            pltpu.SemaphoreType.DMA,
            pltpu.SemaphoreType.DMA,
            pltpu.SemaphoreType.DMA,
            pltpu.SemaphoreType.DMA,
        ],
    )
    def sc_expand(table_hbm, idx_hbm, cvec_hbm, base_hbm, out_hbm,
                  idx_v, gbuf_a, gbuf_b, cbuf_a, cbuf_b, base_v,
                  obuf_a, obuf_b, sem_ga, sem_gb, sem_ca, sem_cb,
                  sem_oa, sem_ob):
        wid = lax.axis_index("s") * _SC_CORES + lax.axis_index("c")
        tok0 = wid * tpw
        pltpu.sync_copy(idx_hbm.at[pl.ds(tok0 * _ND, tpw * _ND)], idx_v)
        pltpu.sync_copy(base_hbm, base_v)

        def start_in(c, gbuf, cbuf, sem_g, sem_c):
            pltpu.async_copy(
                table_hbm.at[idx_v.at[pl.ds(c * rpc, rpc)]], gbuf, sem_g)
            pltpu.async_copy(
                cvec_hbm.at[pl.ds(tok0 + c * _CH2, _CH2)], cbuf, sem_c)

        def wait_in(gbuf, cbuf, sem_g, sem_c):
            pltpu.make_async_copy(
                table_hbm.at[idx_v.at[pl.ds(0, rpc)]], gbuf, sem_g).wait()
            pltpu.make_async_copy(
                cvec_hbm.at[pl.ds(tok0, _CH2)], cbuf, sem_c).wait()

        def wait_out(obuf, sem):
            pltpu.make_async_copy(
                obuf, out_hbm.at[pl.ds(tok0, _CH2)], sem).wait()

        def compute(gbuf, cbuf, obuf):
            @plsc.parallel_loop(0, _D // 16, 1, unroll=2)
            def cbody(dd):
                sl = pl.ds(dd * 16, 16)
                for t in range(_CH2):
                    v = ((gbuf[4 * t + 0, sl] + gbuf[4 * t + 1, sl])
                         + (gbuf[4 * t + 2, sl] + gbuf[4 * t + 3, sl])
                         + cbuf[t, sl])
                    for s in range(_SA):
                        obuf[t, s, sl] = v + base_v[s, sl]

        start_in(0, gbuf_a, cbuf_a, sem_ga, sem_ca)

        def pbody(p, carry):
            c0 = 2 * p
            start_in(c0 + 1, gbuf_b, cbuf_b, sem_gb, sem_cb)
            wait_in(gbuf_a, cbuf_a, sem_ga, sem_ca)

            @pl.when(p > 0)
            def _():
                wait_out(obuf_a, sem_oa)
            compute(gbuf_a, cbuf_a, obuf_a)
            pltpu.async_copy(
                obuf_a, out_hbm.at[pl.ds(tok0 + c0 * _CH2, _CH2)], sem_oa)

            @pl.when(p + 1 < nchunk // 2)
            def _():
                start_in(c0 + 2, gbuf_a, cbuf_a, sem_ga, sem_ca)
            wait_in(gbuf_b, cbuf_b, sem_gb, sem_cb)

            @pl.when(p > 0)
            def _():
                wait_out(obuf_b, sem_ob)
            compute(gbuf_b, cbuf_b, obuf_b)
            pltpu.async_copy(
                obuf_b, out_hbm.at[pl.ds(tok0 + (c0 + 1) * _CH2, _CH2)],
                sem_ob)
            return carry

        lax.fori_loop(0, nchunk // 2, pbody, 0)
        wait_out(obuf_a, sem_oa)
        wait_out(obuf_b, sem_ob)

    return sc_expand


@functools.lru_cache(maxsize=2)
def _sc_expand_cached(ntok):
    return _make_sc_expand(ntok)


_PT = 256


def _pre_body(cont_ref, w_ref, comp_ref, lb_ref, cvec_ref, base_ref):
    cvec_ref[...] = jnp.dot(cont_ref[...], w_ref[...],
                            preferred_element_type=jnp.float32)
    base_ref[...] = (jnp.sum(comp_ref[...], axis=0)
                     + jnp.sum(lb_ref[...], axis=0)[None, :])


def _precompute(cont, w2d, comp, lin_b):
    return pl.pallas_call(
        _pre_body,
        grid=(_NTOK // _PT,),
        in_specs=[
            pl.BlockSpec((_PT, _NC), lambda i: (i, 0)),
            pl.BlockSpec((_NC, _D), lambda i: (0, 0)),
            pl.BlockSpec((_ND + _NC, _SA, _D), lambda i: (0, 0, 0)),
            pl.BlockSpec((_NC, _D), lambda i: (0, 0)),
        ],
        out_specs=[
            pl.BlockSpec((_PT, _D), lambda i: (i, 0)),
            pl.BlockSpec((_SA, _D), lambda i: (0, 0)),
        ],
        out_shape=[
            jax.ShapeDtypeStruct((_NTOK, _D), jnp.float32),
            jax.ShapeDtypeStruct((_SA, _D), jnp.float32),
        ],
        compiler_params=pltpu.CompilerParams(
            dimension_semantics=("arbitrary",)),
    )(cont, w2d, comp, lin_b)


_TT = 256


def _expand_first_body(vec_ref, cont_ref, w_ref, comp_ref, lb_ref, out_ref):
    base = jnp.sum(comp_ref[...], axis=0) + jnp.sum(lb_ref[...], axis=0)[None, :]
    tok = vec_ref[...] + jnp.dot(cont_ref[...], w_ref[...],
                                 preferred_element_type=jnp.float32)
    out_ref[...] = tok[:, None, :] + base[None, :, :]


def _expand_chain_body(prev_ref, vec_ref, cont_ref, w_ref, comp_ref, lb_ref,
                       out_ref):
    del prev_ref
    _expand_first_body(vec_ref, cont_ref, w_ref, comp_ref, lb_ref, out_ref)


def _expand_onehot_body(disc_ref, tbl_ref, cont_ref, w_ref, comp_ref, lb_ref,
                        out_ref):
    tt = disc_ref.shape[0]
    iota = lax.broadcasted_iota(jnp.int32, (tt, _ND, _BINS), 2)
    oh = (iota == disc_ref[...][:, :, None]).astype(jnp.bfloat16)
    vec = jnp.dot(oh.reshape(tt, _ND * _BINS), tbl_ref[...],
                  preferred_element_type=jnp.float32)
    base = jnp.sum(comp_ref[...], axis=0) + jnp.sum(lb_ref[...], axis=0)[None, :]
    tok = vec + jnp.dot(cont_ref[...], w_ref[...],
                        preferred_element_type=jnp.float32)
    out_ref[...] = tok[:, None, :] + base[None, :, :]


def _expand_slice0_onehot(ntok, disc, tbl_bf, cont, w2d, comp, lin_b):
    nblk = ntok // _TT
    return pl.pallas_call(
        _expand_onehot_body,
        grid=(nblk,),
        in_specs=[
            pl.BlockSpec((_TT, _ND), lambda i: (i, 0)),
            pl.BlockSpec((_ND * _BINS, _D), lambda i: (0, 0)),
            pl.BlockSpec((_TT, _NC), lambda i: (i, 0)),
            pl.BlockSpec((_NC, _D), lambda i: (0, 0)),
            pl.BlockSpec((_ND + _NC, _SA, _D), lambda i: (0, 0, 0)),
            pl.BlockSpec((_NC, _D), lambda i: (0, 0)),
        ],
        out_specs=pl.BlockSpec((_TT, _SA, _D), lambda i: (i, 0, 0)),
        out_shape=jax.ShapeDtypeStruct((_NTOK, _SA, _D), jnp.float32),
        compiler_params=pltpu.CompilerParams(
            dimension_semantics=("arbitrary",)),
    )(disc, tbl_bf, cont, w2d, comp, lin_b)


def _expand_slice(tok_off, ntok, prev, vec, cont, w2d, comp, lin_b):
    nblk = ntok // _TT
    data_specs = [
        pl.BlockSpec((_TT, _D), lambda i: (i, 0)),
        pl.BlockSpec((_TT, _NC), lambda i: (i, 0)),
        pl.BlockSpec((_NC, _D), lambda i: (0, 0)),
        pl.BlockSpec((_ND + _NC, _SA, _D), lambda i: (0, 0, 0)),
        pl.BlockSpec((_NC, _D), lambda i: (0, 0)),
    ]
    blk0 = tok_off // _TT
    out_spec = pl.BlockSpec((_TT, _SA, _D),
                            lambda i, _b=blk0: (_b + i, 0, 0))
    out_shape = jax.ShapeDtypeStruct((_NTOK, _SA, _D), jnp.float32)
    params = pltpu.CompilerParams(dimension_semantics=("arbitrary",))
    if prev is None:
        return pl.pallas_call(
            _expand_first_body,
            grid=(nblk,),
            in_specs=data_specs,
            out_specs=out_spec,
            out_shape=out_shape,
            compiler_params=params,
        )(vec, cont, w2d, comp, lin_b)
    return pl.pallas_call(
        _expand_chain_body,
        grid=(nblk,),
        in_specs=[pl.BlockSpec(memory_space=pl.ANY)] + data_specs,
        out_specs=out_spec,
        out_shape=out_shape,
        input_output_aliases={0: 0},
        compiler_params=params,
    )(prev, vec, cont, w2d, comp, lin_b)


def kernel(discrete_actions, continuous_actions, emb_tables, lin_w, lin_b,
           component_tokens):
    table = emb_tables.reshape(_ND * _BINS, _D)
    idx = (discrete_actions.reshape(_NTOK, _ND).astype(jnp.int32)
           + (jnp.arange(_ND, dtype=jnp.int32) * _BINS)[None, :]).reshape(-1)
    cont = continuous_actions.reshape(_NTOK, _NC)
    w2d = lin_w[:, :, 0]
    comp = component_tokens.reshape(_ND + _NC, _SA, _D)

    slices = (3072, 512, 512)
    disc2d = discrete_actions.reshape(_NTOK, _ND).astype(jnp.int32)
    tbl_bf = table.astype(jnp.bfloat16)
    vecs = []
    off = slices[0]
    for n in slices[1:]:
        vecs.append((off, n, _sc_gather_sum_cached(n)(
            table, idx[off * _ND:(off + n) * _ND])))
        off += n
    out = _expand_slice0_onehot(slices[0], disc2d[:slices[0]], tbl_bf,
                                cont[:slices[0]], w2d, comp, lin_b)
    for off, n, vec in vecs:
        out = _expand_slice(off, n, out, vec,
                            cont[off:off + n], w2d, comp, lin_b)
    return out.reshape(_B, _T, _SA, _D)


# speedup vs baseline: 1.4698x; 1.0610x over previous
import functools

import jax
import jax.numpy as jnp
from jax import lax
from jax.experimental import pallas as pl
from jax.experimental.pallas import tpu as pltpu
from jax.experimental.pallas import tpu_sc as plsc

_B = 16
_T = 256
_ND = 4
_NC = 6
_BINS = 256
_SA = 8
_D = 1024
_NTOK = _B * _T

_NSLICE = 4
_SLICE = _NTOK // _NSLICE

_SC_CORES = 2
_SC_SUBCORES = 16
_NW = _SC_CORES * _SC_SUBCORES
_CH = 8
_RPC = _CH * _ND


def _make_sc_gather_sum(ntok):
    tpw = ntok // _NW
    nchunk = tpw // _CH
    mesh = plsc.VectorSubcoreMesh(core_axis_name="c", subcore_axis_name="s")

    @functools.partial(
        pl.kernel,
        mesh=mesh,
        out_type=jax.ShapeDtypeStruct((ntok, _D), jnp.float32),
        scratch_types=[
            pltpu.VMEM((tpw * _ND,), jnp.int32),
            pltpu.VMEM((_RPC, _D), jnp.float32),
            pltpu.VMEM((_RPC, _D), jnp.float32),
            pltpu.VMEM((_CH, _D), jnp.float32),
            pltpu.VMEM((_CH, _D), jnp.float32),
            pltpu.SemaphoreType.DMA,
            pltpu.SemaphoreType.DMA,
            pltpu.SemaphoreType.DMA,
            pltpu.SemaphoreType.DMA,
        ],
    )
    def gather_sum(table_hbm, idx_hbm, out_hbm, idx_v, buf_a, buf_b,
                   acc_a, acc_b, sem_a, sem_b, sem_oa, sem_ob):
        wid = lax.axis_index("s") * _SC_CORES + lax.axis_index("c")
        tok0 = wid * tpw
        pltpu.sync_copy(idx_hbm.at[pl.ds(tok0 * _ND, tpw * _ND)], idx_v)

        def compute(buf, acc):
            @plsc.parallel_loop(0, _CH * 16, 1, unroll=4)
            def cbody(i):
                t = i >> 4
                dd = i & 15
                for u in range(4):
                    sl = pl.ds(dd * 64 + u * 16, 16)
                    acc[t, sl] = ((buf[4 * t + 0, sl] + buf[4 * t + 1, sl])
                                  + (buf[4 * t + 2, sl] + buf[4 * t + 3, sl]))

        def wait_gather(buf, sem):
            pltpu.make_async_copy(
                table_hbm.at[idx_v.at[pl.ds(0, _RPC)]], buf, sem).wait()

        def wait_out(acc, sem):
            pltpu.make_async_copy(
                acc, out_hbm.at[pl.ds(tok0, _CH)], sem).wait()

        pltpu.async_copy(table_hbm.at[idx_v.at[pl.ds(0, _RPC)]], buf_a, sem_a)

        def pbody(p, carry):
            c0 = 2 * p
            pltpu.async_copy(
                table_hbm.at[idx_v.at[pl.ds((c0 + 1) * _RPC, _RPC)]],
                buf_b, sem_b)
            wait_gather(buf_a, sem_a)

            @pl.when(p > 0)
            def _():
                wait_out(acc_a, sem_oa)
            compute(buf_a, acc_a)
            pltpu.async_copy(
                acc_a, out_hbm.at[pl.ds(tok0 + c0 * _CH, _CH)], sem_oa)

            @pl.when(p + 1 < nchunk // 2)
            def _():
                pltpu.async_copy(
                    table_hbm.at[idx_v.at[pl.ds((c0 + 2) * _RPC, _RPC)]],
                    buf_a, sem_a)
            wait_gather(buf_b, sem_b)

            @pl.when(p > 0)
            def _():
                wait_out(acc_b, sem_ob)
            compute(buf_b, acc_b)
            pltpu.async_copy(
                acc_b, out_hbm.at[pl.ds(tok0 + (c0 + 1) * _CH, _CH)], sem_ob)
            return carry

        lax.fori_loop(0, nchunk // 2, pbody, 0)
        wait_out(acc_a, sem_oa)
        wait_out(acc_b, sem_ob)

    return gather_sum


@functools.lru_cache(maxsize=None)
def _sc_gather_sum_cached(ntok):
    return _make_sc_gather_sum(ntok)


_CH2 = 4


def _make_sc_expand(ntok):
    tpw = ntok // _NW
    nchunk = tpw // _CH2
    rpc = _CH2 * _ND
    mesh = plsc.VectorSubcoreMesh(core_axis_name="c", subcore_axis_name="s")

    @functools.partial(
        pl.kernel,
        mesh=mesh,
        out_type=jax.ShapeDtypeStruct((ntok, _SA, _D), jnp.float32),
        scratch_types=[
            pltpu.VMEM((tpw * _ND,), jnp.int32),
            pltpu.VMEM((rpc, _D), jnp.float32),
            pltpu.VMEM((rpc, _D), jnp.float32),
            pltpu.VMEM((_CH2, _D), jnp.float32),
            pltpu.VMEM((_CH2, _D), jnp.float32),
            pltpu.VMEM((_SA, _D), jnp.float32),
            pltpu.VMEM((_CH2, _SA, _D), jnp.float32),
            pltpu.VMEM((_CH2, _SA, _D), jnp.float32),
            pltpu.SemaphoreType.DMA,
            pltpu.SemaphoreType.DMA,
            pltpu.SemaphoreType.DMA,
            pltpu.SemaphoreType.DMA,
            pltpu.SemaphoreType.DMA,
            pltpu.SemaphoreType.DMA,
        ],
    )
    def sc_expand(table_hbm, idx_hbm, cvec_hbm, base_hbm, out_hbm,
                  idx_v, gbuf_a, gbuf_b, cbuf_a, cbuf_b, base_v,
                  obuf_a, obuf_b, sem_ga, sem_gb, sem_ca, sem_cb,
                  sem_oa, sem_ob):
        wid = lax.axis_index("s") * _SC_CORES + lax.axis_index("c")
        tok0 = wid * tpw
        pltpu.sync_copy(idx_hbm.at[pl.ds(tok0 * _ND, tpw * _ND)], idx_v)
        pltpu.sync_copy(base_hbm, base_v)

        def start_in(c, gbuf, cbuf, sem_g, sem_c):
            pltpu.async_copy(
                table_hbm.at[idx_v.at[pl.ds(c * rpc, rpc)]], gbuf, sem_g)
            pltpu.async_copy(
                cvec_hbm.at[pl.ds(tok0 + c * _CH2, _CH2)], cbuf, sem_c)

        def wait_in(gbuf, cbuf, sem_g, sem_c):
            pltpu.make_async_copy(
                table_hbm.at[idx_v.at[pl.ds(0, rpc)]], gbuf, sem_g).wait()
            pltpu.make_async_copy(
                cvec_hbm.at[pl.ds(tok0, _CH2)], cbuf, sem_c).wait()

        def wait_out(obuf, sem):
            pltpu.make_async_copy(
                obuf, out_hbm.at[pl.ds(tok0, _CH2)], sem).wait()

        def compute(gbuf, cbuf, obuf):
            @plsc.parallel_loop(0, _D // 16, 1, unroll=2)
            def cbody(dd):
                sl = pl.ds(dd * 16, 16)
                for t in range(_CH2):
                    v = ((gbuf[4 * t + 0, sl] + gbuf[4 * t + 1, sl])
                         + (gbuf[4 * t + 2, sl] + gbuf[4 * t + 3, sl])
                         + cbuf[t, sl])
                    for s in range(_SA):
                        obuf[t, s, sl] = v + base_v[s, sl]

        start_in(0, gbuf_a, cbuf_a, sem_ga, sem_ca)

        def pbody(p, carry):
            c0 = 2 * p
            start_in(c0 + 1, gbuf_b, cbuf_b, sem_gb, sem_cb)
            wait_in(gbuf_a, cbuf_a, sem_ga, sem_ca)

            @pl.when(p > 0)
            def _():
                wait_out(obuf_a, sem_oa)
            compute(gbuf_a, cbuf_a, obuf_a)
            pltpu.async_copy(
                obuf_a, out_hbm.at[pl.ds(tok0 + c0 * _CH2, _CH2)], sem_oa)

            @pl.when(p + 1 < nchunk // 2)
            def _():
                start_in(c0 + 2, gbuf_a, cbuf_a, sem_ga, sem_ca)
            wait_in(gbuf_b, cbuf_b, sem_gb, sem_cb)

            @pl.when(p > 0)
            def _():
                wait_out(obuf_b, sem_ob)
            compute(gbuf_b, cbuf_b, obuf_b)
            pltpu.async_copy(
                obuf_b, out_hbm.at[pl.ds(tok0 + (c0 + 1) * _CH2, _CH2)],
                sem_ob)
            return carry

        lax.fori_loop(0, nchunk // 2, pbody, 0)
        wait_out(obuf_a, sem_oa)
        wait_out(obuf_b, sem_ob)

    return sc_expand


@functools.lru_cache(maxsize=2)
def _sc_expand_cached(ntok):
    return _make_sc_expand(ntok)


_PT = 256


def _pre_body(cont_ref, w_ref, comp_ref, lb_ref, cvec_ref, base_ref):
    cvec_ref[...] = jnp.dot(cont_ref[...], w_ref[...],
                            preferred_element_type=jnp.float32)
    base_ref[...] = (jnp.sum(comp_ref[...], axis=0)
                     + jnp.sum(lb_ref[...], axis=0)[None, :])


def _precompute(cont, w2d, comp, lin_b):
    return pl.pallas_call(
        _pre_body,
        grid=(_NTOK // _PT,),
        in_specs=[
            pl.BlockSpec((_PT, _NC), lambda i: (i, 0)),
            pl.BlockSpec((_NC, _D), lambda i: (0, 0)),
            pl.BlockSpec((_ND + _NC, _SA, _D), lambda i: (0, 0, 0)),
            pl.BlockSpec((_NC, _D), lambda i: (0, 0)),
        ],
        out_specs=[
            pl.BlockSpec((_PT, _D), lambda i: (i, 0)),
            pl.BlockSpec((_SA, _D), lambda i: (0, 0)),
        ],
        out_shape=[
            jax.ShapeDtypeStruct((_NTOK, _D), jnp.float32),
            jax.ShapeDtypeStruct((_SA, _D), jnp.float32),
        ],
        compiler_params=pltpu.CompilerParams(
            dimension_semantics=("arbitrary",)),
    )(cont, w2d, comp, lin_b)


_TT = 256


def _expand_first_body(vec_ref, cont_ref, w_ref, comp_ref, lb_ref, out_ref):
    base = jnp.sum(comp_ref[...], axis=0) + jnp.sum(lb_ref[...], axis=0)[None, :]
    tok = vec_ref[...] + jnp.dot(cont_ref[...], w_ref[...],
                                 preferred_element_type=jnp.float32)
    out_ref[...] = tok[:, None, :] + base[None, :, :]


def _expand_chain_body(prev_ref, vec_ref, cont_ref, w_ref, comp_ref, lb_ref,
                       out_ref):
    del prev_ref
    _expand_first_body(vec_ref, cont_ref, w_ref, comp_ref, lb_ref, out_ref)


def _expand_onehot_body(disc_ref, tbl_ref, cont_ref, w_ref, comp_ref, lb_ref,
                        out_ref):
    tt = disc_ref.shape[0]
    iota = lax.broadcasted_iota(jnp.int32, (tt, _ND, _BINS), 2)
    oh = (iota == disc_ref[...][:, :, None]).astype(jnp.bfloat16)
    vec = jnp.dot(oh.reshape(tt, _ND * _BINS), tbl_ref[...],
                  preferred_element_type=jnp.float32)
    base = jnp.sum(comp_ref[...], axis=0) + jnp.sum(lb_ref[...], axis=0)[None, :]
    tok = vec + jnp.dot(cont_ref[...], w_ref[...],
                        preferred_element_type=jnp.float32)
    out_ref[...] = tok[:, None, :] + base[None, :, :]


def _expand_slice0_onehot(ntok, disc, tbl_bf, cont, w2d, comp, lin_b):
    nblk = ntok // _TT
    return pl.pallas_call(
        _expand_onehot_body,
        grid=(nblk,),
        in_specs=[
            pl.BlockSpec((_TT, _ND), lambda i: (i, 0)),
            pl.BlockSpec((_ND * _BINS, _D), lambda i: (0, 0)),
            pl.BlockSpec((_TT, _NC), lambda i: (i, 0)),
            pl.BlockSpec((_NC, _D), lambda i: (0, 0)),
            pl.BlockSpec((_ND + _NC, _SA, _D), lambda i: (0, 0, 0)),
            pl.BlockSpec((_NC, _D), lambda i: (0, 0)),
        ],
        out_specs=pl.BlockSpec((_TT, _SA, _D), lambda i: (i, 0, 0)),
        out_shape=jax.ShapeDtypeStruct((_NTOK, _SA, _D), jnp.float32),
        compiler_params=pltpu.CompilerParams(
            dimension_semantics=("arbitrary",)),
    )(disc, tbl_bf, cont, w2d, comp, lin_b)


def _expand_slice(tok_off, ntok, prev, vec, cont, w2d, comp, lin_b):
    nblk = ntok // _TT
    data_specs = [
        pl.BlockSpec((_TT, _D), lambda i: (i, 0)),
        pl.BlockSpec((_TT, _NC), lambda i: (i, 0)),
        pl.BlockSpec((_NC, _D), lambda i: (0, 0)),
        pl.BlockSpec((_ND + _NC, _SA, _D), lambda i: (0, 0, 0)),
        pl.BlockSpec((_NC, _D), lambda i: (0, 0)),
    ]
    blk0 = tok_off // _TT
    out_spec = pl.BlockSpec((_TT, _SA, _D),
                            lambda i, _b=blk0: (_b + i, 0, 0))
    out_shape = jax.ShapeDtypeStruct((_NTOK, _SA, _D), jnp.float32)
    params = pltpu.CompilerParams(dimension_semantics=("arbitrary",))
    if prev is None:
        return pl.pallas_call(
            _expand_first_body,
            grid=(nblk,),
            in_specs=data_specs,
            out_specs=out_spec,
            out_shape=out_shape,
            compiler_params=params,
        )(vec, cont, w2d, comp, lin_b)
    return pl.pallas_call(
        _expand_chain_body,
        grid=(nblk,),
        in_specs=[pl.BlockSpec(memory_space=pl.ANY)] + data_specs,
        out_specs=out_spec,
        out_shape=out_shape,
        input_output_aliases={0: 0},
        compiler_params=params,
    )(prev, vec, cont, w2d, comp, lin_b)


def kernel(discrete_actions, continuous_actions, emb_tables, lin_w, lin_b,
           component_tokens):
    table = emb_tables.reshape(_ND * _BINS, _D)
    idx = (discrete_actions.reshape(_NTOK, _ND).astype(jnp.int32)
           + (jnp.arange(_ND, dtype=jnp.int32) * _BINS)[None, :]).reshape(-1)
    cont = continuous_actions.reshape(_NTOK, _NC)
    w2d = lin_w[:, :, 0]
    comp = component_tokens.reshape(_ND + _NC, _SA, _D)

    slices = (3072, 1024)
    disc2d = discrete_actions.reshape(_NTOK, _ND).astype(jnp.int32)
    tbl_bf = table.astype(jnp.bfloat16)
    vecs = []
    off = slices[0]
    for n in slices[1:]:
        vecs.append((off, n, _sc_gather_sum_cached(n)(
            table, idx[off * _ND:(off + n) * _ND])))
        off += n
    out = _expand_slice0_onehot(slices[0], disc2d[:slices[0]], tbl_bf,
                                cont[:slices[0]], w2d, comp, lin_b)
    for off, n, vec in vecs:
        out = _expand_slice(off, n, out, vec,
                            cont[off:off + n], w2d, comp, lin_b)
    return out.reshape(_B, _T, _SA, _D)


# speedup vs baseline: 1.4702x; 1.0003x over previous
import functools

import jax
import jax.numpy as jnp
from jax import lax
from jax.experimental import pallas as pl
from jax.experimental.pallas import tpu as pltpu
from jax.experimental.pallas import tpu_sc as plsc

_B = 16
_T = 256
_ND = 4
_NC = 6
_BINS = 256
_SA = 8
_D = 1024
_NTOK = _B * _T

_SC_CORES = 2
_SC_SUBCORES = 16
_NW = _SC_CORES * _SC_SUBCORES
_CH = 8
_RPC = _CH * _ND

_SLICES = (3072, 1024)


def _make_sc_gather_sum(ntok):
    tpw = ntok // _NW
    nchunk = tpw // _CH
    mesh = plsc.VectorSubcoreMesh(core_axis_name="c", subcore_axis_name="s")

    @functools.partial(
        pl.kernel,
        mesh=mesh,
        out_type=jax.ShapeDtypeStruct((ntok, _D), jnp.float32),
        scratch_types=[
            pltpu.VMEM((tpw * _ND,), jnp.int32),
            pltpu.VMEM((_RPC, _D), jnp.float32),
            pltpu.VMEM((_RPC, _D), jnp.float32),
            pltpu.VMEM((_CH, _D), jnp.float32),
            pltpu.VMEM((_CH, _D), jnp.float32),
            pltpu.SemaphoreType.DMA,
            pltpu.SemaphoreType.DMA,
            pltpu.SemaphoreType.DMA,
            pltpu.SemaphoreType.DMA,
        ],
    )
    def gather_sum(table_hbm, idx_hbm, out_hbm, idx_v, buf_a, buf_b,
                   acc_a, acc_b, sem_a, sem_b, sem_oa, sem_ob):
        wid = lax.axis_index("s") * _SC_CORES + lax.axis_index("c")
        tok0 = wid * tpw
        pltpu.sync_copy(idx_hbm.at[pl.ds(tok0 * _ND, tpw * _ND)], idx_v)

        def compute(buf, acc):
            @plsc.parallel_loop(0, _CH * 16, 1, unroll=4)
            def cbody(i):
                t = i >> 4
                dd = i & 15
                for u in range(4):
                    sl = pl.ds(dd * 64 + u * 16, 16)
                    acc[t, sl] = ((buf[4 * t + 0, sl] + buf[4 * t + 1, sl])
                                  + (buf[4 * t + 2, sl] + buf[4 * t + 3, sl]))

        def wait_gather(buf, sem):
            pltpu.make_async_copy(
                table_hbm.at[idx_v.at[pl.ds(0, _RPC)]], buf, sem).wait()

        def wait_out(acc, sem):
            pltpu.make_async_copy(
                acc, out_hbm.at[pl.ds(tok0, _CH)], sem).wait()

        pltpu.async_copy(table_hbm.at[idx_v.at[pl.ds(0, _RPC)]], buf_a, sem_a)

        def pbody(p, carry):
            c0 = 2 * p
            pltpu.async_copy(
                table_hbm.at[idx_v.at[pl.ds((c0 + 1) * _RPC, _RPC)]],
                buf_b, sem_b)
            wait_gather(buf_a, sem_a)

            @pl.when(p > 0)
            def _():
                wait_out(acc_a, sem_oa)
            compute(buf_a, acc_a)
            pltpu.async_copy(
                acc_a, out_hbm.at[pl.ds(tok0 + c0 * _CH, _CH)], sem_oa)

            @pl.when(p + 1 < nchunk // 2)
            def _():
                pltpu.async_copy(
                    table_hbm.at[idx_v.at[pl.ds((c0 + 2) * _RPC, _RPC)]],
                    buf_a, sem_a)
            wait_gather(buf_b, sem_b)

            @pl.when(p > 0)
            def _():
                wait_out(acc_b, sem_ob)
            compute(buf_b, acc_b)
            pltpu.async_copy(
                acc_b, out_hbm.at[pl.ds(tok0 + (c0 + 1) * _CH, _CH)], sem_ob)
            return carry

        lax.fori_loop(0, nchunk // 2, pbody, 0)
        wait_out(acc_a, sem_oa)
        wait_out(acc_b, sem_ob)

    return gather_sum


@functools.lru_cache(maxsize=None)
def _sc_gather_sum_cached(ntok):
    return _make_sc_gather_sum(ntok)


_TT = 256


def _expand_first_body(vec_ref, cont_ref, w_ref, comp_ref, lb_ref, out_ref):
    base = jnp.sum(comp_ref[...], axis=0) + jnp.sum(lb_ref[...], axis=0)[None, :]
    tok = vec_ref[...] + jnp.dot(cont_ref[...], w_ref[...],
                                 preferred_element_type=jnp.float32)
    out_ref[...] = tok[:, None, :] + base[None, :, :]


def _expand_chain_body(prev_ref, vec_ref, cont_ref, w_ref, comp_ref, lb_ref,
                       out_ref):
    del prev_ref
    _expand_first_body(vec_ref, cont_ref, w_ref, comp_ref, lb_ref, out_ref)


def _expand_onehot_body(disc_ref, tbl_ref, cont_ref, w_ref, comp_ref, lb_ref,
                        out_ref):
    tt = disc_ref.shape[0]
    iota = lax.broadcasted_iota(jnp.int32, (tt, _ND, _BINS), 2)
    oh = (iota == disc_ref[...][:, :, None]).astype(jnp.bfloat16)
    vec = jnp.dot(oh.reshape(tt, _ND * _BINS), tbl_ref[...],
                  preferred_element_type=jnp.float32)
    base = jnp.sum(comp_ref[...], axis=0) + jnp.sum(lb_ref[...], axis=0)[None, :]
    tok = vec + jnp.dot(cont_ref[...], w_ref[...],
                        preferred_element_type=jnp.float32)
    out_ref[...] = tok[:, None, :] + base[None, :, :]


def _expand_slice0_onehot(ntok, disc, tbl_bf, cont, w2d, comp, lin_b):
    nblk = ntok // _TT
    return pl.pallas_call(
        _expand_onehot_body,
        grid=(nblk,),
        in_specs=[
            pl.BlockSpec((_TT, _ND), lambda i: (i, 0)),
            pl.BlockSpec((_ND * _BINS, _D), lambda i: (0, 0)),
            pl.BlockSpec((_TT, _NC), lambda i: (i, 0)),
            pl.BlockSpec((_NC, _D), lambda i: (0, 0)),
            pl.BlockSpec((_ND + _NC, _SA, _D), lambda i: (0, 0, 0)),
            pl.BlockSpec((_NC, _D), lambda i: (0, 0)),
        ],
        out_specs=pl.BlockSpec((_TT, _SA, _D), lambda i: (i, 0, 0)),
        out_shape=jax.ShapeDtypeStruct((_NTOK, _SA, _D), jnp.float32),
        compiler_params=pltpu.CompilerParams(
            dimension_semantics=("arbitrary",)),
    )(disc, tbl_bf, cont, w2d, comp, lin_b)


def _expand_slice(tok_off, ntok, prev, vec, cont, w2d, comp, lin_b):
    nblk = ntok // _TT
    data_specs = [
        pl.BlockSpec((_TT, _D), lambda i: (i, 0)),
        pl.BlockSpec((_TT, _NC), lambda i: (i, 0)),
        pl.BlockSpec((_NC, _D), lambda i: (0, 0)),
        pl.BlockSpec((_ND + _NC, _SA, _D), lambda i: (0, 0, 0)),
        pl.BlockSpec((_NC, _D), lambda i: (0, 0)),
    ]
    blk0 = tok_off // _TT
    out_spec = pl.BlockSpec((_TT, _SA, _D),
                            lambda i, _b=blk0: (_b + i, 0, 0))
    return pl.pallas_call(
        _expand_chain_body,
        grid=(nblk,),
        in_specs=[pl.BlockSpec(memory_space=pl.ANY)] + data_specs,
        out_specs=out_spec,
        out_shape=jax.ShapeDtypeStruct((_NTOK, _SA, _D), jnp.float32),
        input_output_aliases={0: 0},
        compiler_params=pltpu.CompilerParams(
            dimension_semantics=("arbitrary",)),
    )(prev, vec, cont, w2d, comp, lin_b)


def kernel(discrete_actions, continuous_actions, emb_tables, lin_w, lin_b,
           component_tokens):
    table = emb_tables.reshape(_ND * _BINS, _D)
    idx = (discrete_actions.reshape(_NTOK, _ND).astype(jnp.int32)
           + (jnp.arange(_ND, dtype=jnp.int32) * _BINS)[None, :]).reshape(-1)
    cont = continuous_actions.reshape(_NTOK, _NC)
    w2d = lin_w[:, :, 0]
    comp = component_tokens.reshape(_ND + _NC, _SA, _D)
    disc2d = discrete_actions.reshape(_NTOK, _ND).astype(jnp.int32)
    tbl_bf = table.astype(jnp.bfloat16)

    vecs = []
    off = _SLICES[0]
    for n in _SLICES[1:]:
        vecs.append((off, n, _sc_gather_sum_cached(n)(
            table, idx[off * _ND:(off + n) * _ND])))
        off += n
    out = _expand_slice0_onehot(_SLICES[0], disc2d[:_SLICES[0]], tbl_bf,
                                cont[:_SLICES[0]], w2d, comp, lin_b)
    for off, n, vec in vecs:
        out = _expand_slice(off, n, out, vec,
                            cont[off:off + n], w2d, comp, lin_b)
    return out.reshape(_B, _T, _SA, _D)
